# Initial kernel scaffold; baseline (speedup 1.0000x reference)
import jax
import jax.numpy as jnp
from jax.experimental import pallas as pl


def kernel(features, edge_index, W_in, b_in, W_conv, W_out, b_out):
    raise NotImplementedError("write your pallas kernel here")


# speedup vs baseline: 4.6534x; 4.6534x over previous
import functools
import math

import jax
import jax.numpy as jnp
from jax import lax
from jax.experimental import pallas as pl
from jax.experimental.pallas import tpu as pltpu
from jax.experimental.pallas import tpu_sc as plsc

_N = 10000
_E = 320000
_D = 128
_N_LAYERS = 4
_ALPHA = 0.1
_LAMDA = 0.5

_NC = 2
_NS = 16
_NW = _NC * _NS

_CH = 128
_CPW = 80
_EPW = _CH * _CPW
_EPAD = _EPW * _NW
_NPAD = 10240
_RPT = _NPAD // _NS

_R = 1024
_GRID = _NPAD // _R


def _sc_agg_body(g_hbm, src_hbm, dst_hbm, zeros_hbm, out_hbm,
                 src_v, dst_v, rows_v, acc, sem):
    c = lax.axis_index("c")
    s = lax.axis_index("s")
    wid = s * _NC + c
    pltpu.sync_copy(zeros_hbm.at[pl.ds(s * _RPT, _RPT)],
                    acc.at[pl.ds(s * _RPT, _RPT)])
    pltpu.sync_copy(src_hbm.at[pl.ds(wid * _CPW, _CPW)], src_v)
    pltpu.sync_copy(dst_hbm.at[pl.ds(wid * _CPW, _CPW)], dst_v)
    plsc.subcore_barrier()

    def chunk(j, carry):
        pltpu.async_copy(g_hbm.at[src_v.at[j]], rows_v, sem).wait()
        pltpu.sync_copy(rows_v, acc.at[dst_v.at[j]], add=True)
        return carry

    lax.fori_loop(0, _CPW, chunk, 0)
    plsc.subcore_barrier()
    pltpu.sync_copy(acc.at[pl.ds(s * _RPT, _RPT)],
                    out_hbm.at[pl.ds(c * _NPAD + s * _RPT, _RPT)])


@jax.jit
def _sc_agg(g, src2, dst2, zeros128):
    fn = pl.kernel(
        _sc_agg_body,
        out_type=jax.ShapeDtypeStruct((2 * _NPAD, _D), jnp.float32),
        mesh=plsc.VectorSubcoreMesh(core_axis_name="c", subcore_axis_name="s"),
        scratch_types=[
            pltpu.VMEM((_CPW, _CH), jnp.int32),
            pltpu.VMEM((_CPW, _CH), jnp.int32),
            pltpu.VMEM((_CH, _D), jnp.float32),
            pltpu.VMEM_SHARED((_NPAD, _D), jnp.float32),
            pltpu.SemaphoreType.DMA,
        ],
    )
    return fn(g, src2, dst2, zeros128)


def _sc_deg_body(dst_hbm, zeros_hbm, ones_hbm, out_hbm,
                 dst_v, ones_v, acc, sem):
    c = lax.axis_index("c")
    s = lax.axis_index("s")
    wid = s * _NC + c
    pltpu.sync_copy(zeros_hbm.at[pl.ds(s * _RPT, _RPT)],
                    acc.at[pl.ds(s * _RPT, _RPT)])
    pltpu.sync_copy(dst_hbm.at[pl.ds(wid * _CPW, _CPW)], dst_v)
    pltpu.sync_copy(ones_hbm, ones_v)
    plsc.subcore_barrier()

    def chunk(j, carry):
        pltpu.sync_copy(ones_v, acc.at[dst_v.at[j]], add=True)
        return carry

    lax.fori_loop(0, _CPW, chunk, 0)
    plsc.subcore_barrier()
    pltpu.sync_copy(acc.at[pl.ds(s * _RPT, _RPT)],
                    out_hbm.at[pl.ds(c * _NPAD + s * _RPT, _RPT)])


@jax.jit
def _sc_degree(dst2, zeros128, ones128):
    fn = pl.kernel(
        _sc_deg_body,
        out_type=jax.ShapeDtypeStruct((2 * _NPAD, _D), jnp.float32),
        mesh=plsc.VectorSubcoreMesh(core_axis_name="c", subcore_axis_name="s"),
        scratch_types=[
            pltpu.VMEM((_CPW, _CH), jnp.int32),
            pltpu.VMEM((_CH, _D), jnp.float32),
            pltpu.VMEM_SHARED((_NPAD, _D), jnp.float32),
            pltpu.SemaphoreType.DMA,
        ],
    )
    return fn(dst2, zeros128, ones128)


def _tc_input_body(f_ref, w_ref, b_ref, degp_ref, h0_ref, g_ref, isd_ref):
    deg = degp_ref[0][:, 0:1] + degp_ref[1][:, 0:1] + 1.0
    isd = lax.rsqrt(deg)
    h = jnp.dot(f_ref[...], w_ref[...], preferred_element_type=jnp.float32)
    h = jnp.maximum(h + b_ref[...], 0.0)
    h0_ref[...] = h
    g_ref[...] = h * isd
    isd_ref[...] = jnp.broadcast_to(isd, (_R, 16))


def _tc_input(features, W_in, b_in, degp):
    return pl.pallas_call(
        _tc_input_body,
        grid=(_GRID,),
        in_specs=[
            pl.BlockSpec((_R, _D), lambda i: (i, 0)),
            pl.BlockSpec((_D, _D), lambda i: (0, 0)),
            pl.BlockSpec((1, _D), lambda i: (0, 0)),
            [pl.BlockSpec((_R, _D), lambda i: (i, 0)),
             pl.BlockSpec((_R, _D), lambda i: (_GRID + i, 0))],
        ],
        out_specs=[
            pl.BlockSpec((_R, _D), lambda i: (i, 0)),
            pl.BlockSpec((_R, _D), lambda i: (i, 0)),
            pl.BlockSpec((_R, 16), lambda i: (i, 0)),
        ],
        out_shape=[
            jax.ShapeDtypeStruct((_NPAD, _D), jnp.float32),
            jax.ShapeDtypeStruct((_NPAD, _D), jnp.float32),
            jax.ShapeDtypeStruct((_NPAD, 16), jnp.float32),
        ],
    )(features, W_in, b_in, [degp, degp])


def _tc_layer_body(beta, p_ref, g_ref, h0_ref, isd_ref, w_ref, gout_ref):
    isd = isd_ref[:, 0:1]
    ssum = p_ref[0][...] + p_ref[1][...] + g_ref[...]
    support = (1.0 - _ALPHA) * (isd * ssum) + _ALPHA * h0_ref[...]
    sw = jnp.dot(support, w_ref[...], preferred_element_type=jnp.float32)
    h = jnp.maximum((1.0 - beta) * support + beta * sw, 0.0)
    gout_ref[...] = h * isd


def _tc_layer(beta, p, g, h0, isd, W):
    return pl.pallas_call(
        functools.partial(_tc_layer_body, beta),
        grid=(_GRID,),
        in_specs=[
            [pl.BlockSpec((_R, _D), lambda i: (i, 0)),
             pl.BlockSpec((_R, _D), lambda i: (_GRID + i, 0))],
            pl.BlockSpec((_R, _D), lambda i: (i, 0)),
            pl.BlockSpec((_R, _D), lambda i: (i, 0)),
            pl.BlockSpec((_R, 16), lambda i: (i, 0)),
            pl.BlockSpec((_D, _D), lambda i: (0, 0)),
        ],
        out_specs=pl.BlockSpec((_R, _D), lambda i: (i, 0)),
        out_shape=jax.ShapeDtypeStruct((_NPAD, _D), jnp.float32),
    )([p, p], g, h0, isd, W)


def _tc_final_body(beta, p_ref, g_ref, h0_ref, isd_ref, w_ref, wout_ref,
                   bout_ref, out_ref):
    isd = isd_ref[:, 0:1]
    ssum = p_ref[0][...] + p_ref[1][...] + g_ref[...]
    support = (1.0 - _ALPHA) * (isd * ssum) + _ALPHA * h0_ref[...]
    sw = jnp.dot(support, w_ref[...], preferred_element_type=jnp.float32)
    h = jnp.maximum((1.0 - beta) * support + beta * sw, 0.0)
    out = jnp.dot(h, wout_ref[...], preferred_element_type=jnp.float32)
    out_ref[...] = out + bout_ref[...]


def _tc_final(beta, p, g, h0, isd, W, W_out, b_out):
    return pl.pallas_call(
        functools.partial(_tc_final_body, beta),
        grid=(_GRID,),
        in_specs=[
            [pl.BlockSpec((_R, _D), lambda i: (i, 0)),
             pl.BlockSpec((_R, _D), lambda i: (_GRID + i, 0))],
            pl.BlockSpec((_R, _D), lambda i: (i, 0)),
            pl.BlockSpec((_R, _D), lambda i: (i, 0)),
            pl.BlockSpec((_R, 16), lambda i: (i, 0)),
            pl.BlockSpec((_D, _D), lambda i: (0, 0)),
            pl.BlockSpec((_D, _D), lambda i: (0, 0)),
            pl.BlockSpec((1, _D), lambda i: (0, 0)),
        ],
        out_specs=pl.BlockSpec((_R, _D), lambda i: (i, 0)),
        out_shape=jax.ShapeDtypeStruct((_N, _D), jnp.float32),
    )([p, p], g, h0, isd, W, W_out, b_out)


def kernel(features, edge_index, W_in, b_in, W_conv, W_out, b_out):
    src = edge_index[0]
    dst = edge_index[1]
    pad = jnp.full((_EPAD - _E,), _N, dtype=jnp.int32)
    src2 = jnp.concatenate([src, pad]).reshape(_EPAD // _CH, _CH)
    dst2 = jnp.concatenate([dst, pad]).reshape(_EPAD // _CH, _CH)
    zeros128 = jnp.zeros((_NPAD, _D), jnp.float32)
    ones128 = jnp.ones((_CH, _D), jnp.float32)
    fpad = jnp.zeros((_NPAD - _N, _D), jnp.float32)
    fpadded = jnp.concatenate([features, fpad], axis=0)

    degp = _sc_degree(dst2, zeros128, ones128)
    h0, g, isd = _tc_input(fpadded, W_in, b_in.reshape(1, _D), degp)
    out = None
    for l in range(_N_LAYERS):
        p = _sc_agg(g, src2, dst2, zeros128)
        beta = math.log(_LAMDA / (l + 1) + 1.0)
        if l < _N_LAYERS - 1:
            g = _tc_layer(beta, p, g, h0, isd, W_conv[l])
        else:
            out = _tc_final(beta, p, g, h0, isd, W_conv[l], W_out,
                            b_out.reshape(1, _D))
    return out
